# baseline (speedup 1.0000x reference)
import functools

import jax
import jax.numpy as jnp
from jax.experimental import pallas as pl
from jax.experimental.pallas import tpu as pltpu

NEG_INF = float("-inf")


def _fused_body(labels_ref, h_ref, w1_ref, b1_ref, w2_ref, b2_ref,
                coarse_out, fine_out, flat_out, acc,
                *, n_h, n_k, nf0, nf1, nc):
    k = pl.program_id(1)
    head = k // n_h
    chunk = k % n_h
    hh = h_ref[...].astype(jnp.bfloat16)

    z = jax.lax.dot_general(
        hh, w1_ref[0], (((1,), (1,)), ((), ())),
        preferred_element_type=jnp.float32)
    z = z + b1_ref[0]
    z = (z * 0.5 * (1.0 + jax.lax.erf(z * 0.7071067811865476))
         ).astype(jnp.bfloat16)
    part = jax.lax.dot_general(
        z, w2_ref[0], (((1,), (1,)), ((), ())),
        preferred_element_type=jnp.float32)

    col = head * nf1

    @pl.when(chunk == 0)
    def _():
        acc[:, pl.ds(col, nf1)] = part

    @pl.when(chunk != 0)
    def _():
        acc[:, pl.ds(col, nf1)] += part

    @pl.when(k == n_k - 1)
    def _():
        res = acc[...] + b2_ref[...]
        coarse_out[...] = res[:, :nc]
        l0 = res[:, nf1:nf1 + nf0]
        l1 = res[:, 2 * nf1:3 * nf1]
        mask = labels_ref[...] == 0
        neg = jnp.float32(NEG_INF)
        pad0 = jnp.concatenate(
            [l0, jnp.full((l0.shape[0], nf1 - nf0), neg, jnp.float32)], axis=1)
        fine_out[...] = jnp.where(mask, pad0, l1)
        flat_out[...] = jnp.concatenate(
            [jnp.where(mask, l0, neg), jnp.where(mask, neg, l1)], axis=1)


def kernel(h, coarse_labels, Wc1, bc1, Wc2, bc2,
           Wf0_1, bf0_1, Wf0_2, bf0_2, Wf1_1, bf1_1, Wf1_2, bf1_2):
    B, IN = h.shape
    H = Wc1.shape[0]
    NC = Wc2.shape[0]
    NF0 = Wf0_2.shape[0]
    NF1 = Wf1_2.shape[0]
    bt = min(512, B)
    hc = min(512, H)
    n_b = B // bt
    n_h = H // hc
    n_k = 3 * n_h

    bf = jnp.bfloat16
    w1_all = jnp.concatenate([Wc1, Wf0_1, Wf1_1], axis=0).astype(bf) \
        .reshape(n_k, hc, IN)
    b1_all = jnp.concatenate([bc1, bf0_1, bf1_1]).reshape(n_k, 1, hc)
    zp = functools.partial(jnp.zeros, dtype=jnp.float32)
    w2_pad = jnp.concatenate([
        jnp.concatenate([Wc2, zp((NF1 - NC, H))], 0),
        jnp.concatenate([Wf0_2, zp((NF1 - NF0, H))], 0),
        Wf1_2], axis=0).reshape(3, NF1, n_h, hc)
    w2_all = jnp.transpose(w2_pad, (0, 2, 1, 3)).reshape(n_k, NF1, hc) \
        .astype(bf)
    b2_all = jnp.concatenate([
        bc2, zp((NF1 - NC,)), bf0_2, zp((NF1 - NF0,)), bf1_2,
    ]).reshape(1, 3 * NF1)
    labels2 = coarse_labels.reshape(B, 1)

    in_specs = [
        pl.BlockSpec((bt, 1), lambda b, k: (b, 0)),
        pl.BlockSpec((bt, IN), lambda b, k: (b, 0)),
        pl.BlockSpec((1, hc, IN), lambda b, k: (k, 0, 0)),
        pl.BlockSpec((1, 1, hc), lambda b, k: (k, 0, 0)),
        pl.BlockSpec((1, NF1, hc), lambda b, k: (k, 0, 0)),
        pl.BlockSpec((1, 3 * NF1), lambda b, k: (0, 0)),
    ]

    def out_spec(n):
        return pl.BlockSpec((bt, n), lambda b, k: (b, 0))

    coarse, fine, flat = pl.pallas_call(
        functools.partial(_fused_body, n_h=n_h, n_k=n_k,
                          nf0=NF0, nf1=NF1, nc=NC),
        grid=(n_b, n_k),
        in_specs=in_specs,
        out_specs=[out_spec(NC), out_spec(NF1), out_spec(NF0 + NF1)],
        out_shape=[
            jax.ShapeDtypeStruct((B, NC), jnp.float32),
            jax.ShapeDtypeStruct((B, NF1), jnp.float32),
            jax.ShapeDtypeStruct((B, NF0 + NF1), jnp.float32),
        ],
        scratch_shapes=[
            pltpu.VMEM((bt, 3 * NF1), jnp.float32),
        ],
    )(labels2, h, w1_all, b1_all, w2_all, b2_all)
    return (coarse, fine, flat)


# speedup vs baseline: 1.2764x; 1.2764x over previous
import functools

import jax
import jax.numpy as jnp
from jax.experimental import pallas as pl
from jax.experimental.pallas import tpu as pltpu

NEG_INF = float("-inf")


def _fused_body(labels_ref, h_ref,
                wc1_ref, wf01_ref, wf11_ref,
                bc1_ref, bf01_ref, bf11_ref,
                wc2_ref, wf02_ref, wf12_ref,
                bc2_ref, bf02_ref, bf12_ref,
                coarse_out, fine_out, flat_out,
                acc_c, acc_0, acc_1,
                *, n_h, bt):
    hstep = pl.program_id(0)
    b = pl.program_id(1)
    rows = pl.ds(b * bt, bt)
    hh = h_ref[...].astype(jnp.bfloat16)

    def head_partial(w1_ref, b1_ref, w2_ref):
        z = jax.lax.dot_general(
            hh, w1_ref[...], (((1,), (1,)), ((), ())),
            preferred_element_type=jnp.float32)
        z = z + b1_ref[...]
        z = (z * 0.5 * (1.0 + jax.lax.erf(z * 0.7071067811865476))
             ).astype(jnp.bfloat16)
        return jax.lax.dot_general(
            z, w2_ref[...], (((1,), (1,)), ((), ())),
            preferred_element_type=jnp.float32)

    pc = head_partial(wc1_ref, bc1_ref, wc2_ref)
    p0 = head_partial(wf01_ref, bf01_ref, wf02_ref)
    p1 = head_partial(wf11_ref, bf11_ref, wf12_ref)

    @pl.when(hstep == 0)
    def _():
        acc_c[rows, :] = pc
        acc_0[rows, :] = p0
        acc_1[rows, :] = p1

    @pl.when(hstep != 0)
    def _():
        acc_c[rows, :] += pc
        acc_0[rows, :] += p0
        acc_1[rows, :] += p1

    @pl.when(hstep == n_h - 1)
    def _():
        coarse_out[...] = acc_c[rows, :] + bc2_ref[...]
        l0 = acc_0[rows, :] + bf02_ref[...]
        l1 = acc_1[rows, :] + bf12_ref[...]
        nf0 = l0.shape[1]
        nf1 = l1.shape[1]
        mask = labels_ref[...] == 0
        neg = jnp.float32(NEG_INF)
        pad0 = jnp.concatenate(
            [l0, jnp.full((l0.shape[0], nf1 - nf0), neg, jnp.float32)], axis=1)
        fine_out[...] = jnp.where(mask, pad0, l1)
        flat_out[...] = jnp.concatenate(
            [jnp.where(mask, l0, neg), jnp.where(mask, neg, l1)], axis=1)


def kernel(h, coarse_labels, Wc1, bc1, Wc2, bc2,
           Wf0_1, bf0_1, Wf0_2, bf0_2, Wf1_1, bf1_1, Wf1_2, bf1_2):
    B, IN = h.shape
    H = Wc1.shape[0]
    NC = Wc2.shape[0]
    NF0 = Wf0_2.shape[0]
    NF1 = Wf1_2.shape[0]
    bt = min(512, B)
    hc = min(512, H)
    n_b = B // bt
    n_h = H // hc

    bf = jnp.bfloat16
    w1s = [Wc1.astype(bf), Wf0_1.astype(bf), Wf1_1.astype(bf)]
    b1s = [bc1.reshape(1, H), bf0_1.reshape(1, H), bf1_1.reshape(1, H)]
    w2s = [Wc2.astype(bf), Wf0_2.astype(bf), Wf1_2.astype(bf)]
    b2s = [bc2.reshape(1, NC), bf0_2.reshape(1, NF0), bf1_2.reshape(1, NF1)]
    labels2 = coarse_labels.reshape(B, 1)

    w1_spec = pl.BlockSpec((hc, IN), lambda hs, b: (hs, 0))
    b1_spec = pl.BlockSpec((1, hc), lambda hs, b: (0, hs))

    def w2_spec(n):
        return pl.BlockSpec((n, hc), lambda hs, b: (0, hs))

    def b2_spec(n):
        return pl.BlockSpec((1, n), lambda hs, b: (0, 0))

    def out_spec(n):
        return pl.BlockSpec((bt, n), lambda hs, b: (b, 0))

    in_specs = [
            pl.BlockSpec((bt, 1), lambda hs, b: (b, 0)),
            pl.BlockSpec((bt, IN), lambda hs, b: (b, 0)),
            w1_spec, w1_spec, w1_spec,
            b1_spec, b1_spec, b1_spec,
            w2_spec(NC), w2_spec(NF0), w2_spec(NF1),
            b2_spec(NC), b2_spec(NF0), b2_spec(NF1),
    ]
    out_specs = [out_spec(NC), out_spec(NF1), out_spec(NF0 + NF1)]

    out_shapes = [
        jax.ShapeDtypeStruct((B, NC), jnp.float32),
        jax.ShapeDtypeStruct((B, NF1), jnp.float32),
        jax.ShapeDtypeStruct((B, NF0 + NF1), jnp.float32),
    ]

    coarse, fine, flat = pl.pallas_call(
        functools.partial(_fused_body, n_h=n_h, bt=bt),
        grid=(n_h, n_b),
        in_specs=in_specs,
        out_specs=out_specs,
        out_shape=out_shapes,
        scratch_shapes=[
            pltpu.VMEM((B, NC), jnp.float32),
            pltpu.VMEM((B, NF0), jnp.float32),
            pltpu.VMEM((B, NF1), jnp.float32),
        ],
    )(labels2, h, *w1s, *b1s, *w2s, *b2s)
    return (coarse, fine, flat)
